# Initial kernel scaffold; baseline (speedup 1.0000x reference)
import jax
import jax.numpy as jnp
from jax.experimental import pallas as pl


def kernel(input, masks, target_hists, target_mins, target_maxs):
    raise NotImplementedError("write your pallas kernel here")


# speedup vs baseline: 1357.2340x; 1357.2340x over previous
import jax
import jax.numpy as jnp
from jax import lax
from jax.experimental import pallas as pl
from jax.experimental.pallas import tpu as pltpu
from jax.experimental.pallas import tpu_sc as plsc

_NBINS = 256
_C = 96
_N = 512 * 512
_J = 2
_KEYBITS = 11
_NB = 1 << _KEYBITS
_NLANE = 16
_NARR = 3
_TBL = _NARR * _NLANE * _NB
_TASKS = _J * _C
_NWORK = 32
_TPW = _TASKS // _NWORK
_CH = 4096
_NCHUNK = _N // _CH


def _sc_body(in_hbm, masks_hbm, out_hbm, tbl, inbuf, mbuf):
    wid = lax.axis_index("s") * 2 + lax.axis_index("c")
    laneoff = lax.iota(jnp.int32, 16) * _NB
    ones = jnp.ones((16,), jnp.float32)
    signbit = jnp.int32(-2147483648)

    @pl.loop(0, _TPW)
    def _task(t):
        task = wid * _TPW + t
        j = jnp.where(task >= _C, 1, 0)
        c = task - j * _C

        @pl.loop(0, _TBL // 16)
        def _zero(i):
            tbl[pl.ds(i * 16, 16)] = jnp.zeros((16,), jnp.float32)

        @pl.loop(0, _NCHUNK)
        def _chunk(s):
            off = s * _CH
            pltpu.sync_copy(in_hbm.at[c, pl.ds(off, _CH)], inbuf)
            pltpu.sync_copy(masks_hbm.at[j, pl.ds(off, _CH)], mbuf)

            @pl.loop(0, _CH // 16)
            def _vec(i):
                v = inbuf[pl.ds(i * 16, 16)]
                m = mbuf[pl.ds(i * 16, 16)]
                x = m * v
                bi = lax.bitcast_convert_type(x, jnp.int32)
                sgn = jnp.right_shift(bi, 31)
                key = jnp.bitwise_xor(bi, jnp.bitwise_or(sgn, signbit))
                b = lax.shift_right_logical(key, 32 - _KEYBITS)
                idx = laneoff + b
                plsc.addupdate_scatter(tbl, [idx], ones)
                plsc.addupdate_scatter(tbl, [idx + _NLANE * _NB], x)
                plsc.addupdate_scatter(tbl, [idx + 2 * _NLANE * _NB], x * x)

        pltpu.sync_copy(tbl, out_hbm.at[task])


import functools


@functools.cache
def _sc_hist_kernel():
    return pl.kernel(
        _sc_body,
        out_type=jax.ShapeDtypeStruct((_TASKS, _TBL), jnp.float32),
        mesh=plsc.VectorSubcoreMesh(core_axis_name="c", subcore_axis_name="s"),
        scratch_types=[
            pltpu.VMEM((_TBL,), jnp.float32),
            pltpu.VMEM((_CH,), jnp.float32),
            pltpu.VMEM((_CH,), jnp.float32),
        ],
        compiler_params=pltpu.CompilerParams(needs_layout_passes=False),
    )


def _dotg(a, b_rowvec):
    return lax.dot_general(
        a, b_rowvec, (((1,), (1,)), ((), ())),
        precision=lax.Precision.HIGHEST,
        preferred_element_type=jnp.float32)


def _fin_body(hist_ref, th_ref, tmin_ref, tmax_ref, u2048_ref, i256_ref,
              u256_ref, out_ref):
    cnt = jnp.sum(hist_ref[0, 0], axis=0, keepdims=True)
    vsum = jnp.sum(hist_ref[0, 1], axis=0, keepdims=True)
    vsq = jnp.sum(hist_ref[0, 2], axis=0, keepdims=True)
    sumsq = jnp.sum(vsq)

    cc = lax.dot_general(cnt, u2048_ref[...], (((1,), (0,)), ((), ())),
                         precision=lax.Precision.HIGHEST,
                         preferred_element_type=jnp.float32)
    ccp = cc - cnt

    th = th_ref[0]
    cdf = lax.dot_general(th, u256_ref[...], (((1,), (0,)), ((), ())),
                          precision=lax.Precision.HIGHEST,
                          preferred_element_type=jnp.float32)
    total = jnp.maximum(jnp.max(cdf), 1e-12)
    cdfs = cdf / total * jnp.float32(_N)
    i_lane = lax.broadcasted_iota(jnp.int32, (1, _NBINS), 1).astype(jnp.float32)
    r = jnp.floor(cdfs)
    r = jnp.clip(r, 0.0, jnp.float32(_N))
    r = jnp.where(i_lane == jnp.float32(_NBINS - 1), jnp.float32(_N), r)
    rt = _dotg(i256_ref[...], r)

    mlt = (cc < rt).astype(jnp.float32)
    mle = (ccp < rt).astype(jnp.float32)
    below_cnt = _dotg(mlt, cnt)
    below_sum = _dotg(mlt, vsum)
    at_cnt = _dotg(mle, cnt) - below_cnt
    at_sum = _dotg(mle, vsum) - below_sum
    at_sq = _dotg(mle, vsq) - _dotg(mlt, vsq)

    m = rt - below_cnt
    ac = jnp.maximum(at_cnt, 1.0)
    mu = at_sum / ac
    var = jnp.maximum(at_sq / ac - mu * mu, 0.0)
    w = jnp.sqrt(12.0 * var)
    s = below_sum + m * mu - 0.5 * w * m * (1.0 - m / ac)

    tmin = tmin_ref[0, 0, 0]
    tmax = tmax_ref[0, 0, 0]
    i_sub = lax.broadcasted_iota(jnp.int32, (_NBINS, 1), 0).astype(jnp.float32)
    scale = (tmax - tmin) / jnp.float32(_NBINS - 1)
    tv = i_sub * scale + tmin
    last = i_sub == jnp.float32(_NBINS - 1)
    tnext = jnp.where(last, 0.0, (i_sub + 1.0) * scale + tmin)
    tnext2 = jnp.where(last, 0.0, tnext * tnext)
    cross = jnp.sum(s * (tv - tnext))
    sum_t2 = jnp.sum(rt * (tv * tv - tnext2))
    out_ref[0] = (sumsq - 2.0 * cross + sum_t2).reshape(1, 1)


def kernel(input, masks, target_hists, target_mins, target_maxs):
    inp2 = input.reshape(_C, _N)
    m2 = masks.reshape(_J, _N)
    hist = _sc_hist_kernel()(inp2, m2)
    hist4 = hist.reshape(_TASKS, _NARR, _NLANE, _NB)

    th = target_hists.reshape(_TASKS, 1, _NBINS)
    tmin = target_mins.reshape(_TASKS, 1, 1)
    tmax = target_maxs.reshape(_TASKS, 1, 1)

    k2 = lax.broadcasted_iota(jnp.int32, (_NB, _NB), 0)
    b2 = lax.broadcasted_iota(jnp.int32, (_NB, _NB), 1)
    u2048 = (k2 <= b2).astype(jnp.float32)
    k1 = lax.broadcasted_iota(jnp.int32, (_NBINS, _NBINS), 0)
    b1 = lax.broadcasted_iota(jnp.int32, (_NBINS, _NBINS), 1)
    u256 = (k1 <= b1).astype(jnp.float32)
    i256 = (k1 == b1).astype(jnp.float32)

    parts = pl.pallas_call(
        _fin_body,
        grid=(_TASKS,),
        in_specs=[
            pl.BlockSpec((1, _NARR, _NLANE, _NB), lambda i: (i, 0, 0, 0)),
            pl.BlockSpec((1, 1, _NBINS), lambda i: (i, 0, 0)),
            pl.BlockSpec((1, 1, 1), lambda i: (i, 0, 0)),
            pl.BlockSpec((1, 1, 1), lambda i: (i, 0, 0)),
            pl.BlockSpec((_NB, _NB), lambda i: (0, 0)),
            pl.BlockSpec((_NBINS, _NBINS), lambda i: (0, 0)),
            pl.BlockSpec((_NBINS, _NBINS), lambda i: (0, 0)),
        ],
        out_specs=pl.BlockSpec((1, 1, 1), lambda i: (i, 0, 0)),
        out_shape=jax.ShapeDtypeStruct((_TASKS, 1, 1), jnp.float32),
    )(hist4, th, tmin, tmax, u2048, i256, u256)

    return (0.01 / (_C * _N)) * jnp.sum(parts)


# speedup vs baseline: 1402.7666x; 1.0335x over previous
import functools

import jax
import jax.numpy as jnp
from jax import lax
from jax.experimental import pallas as pl
from jax.experimental.pallas import tpu as pltpu
from jax.experimental.pallas import tpu_sc as plsc

_NBINS = 256
_C = 96
_N = 512 * 512
_J = 2
_KEYBITS = 11
_NB = 1 << _KEYBITS
_NLANE = 16
_NARR = 3
_TBL = _NARR * _NLANE * _NB
_RED = _NARR * _NB
_TASKS = _J * _C
_NWORK = 32
_TPW = _TASKS // _NWORK
_CH = 4096
_NCHUNK = _N // _CH


def _sc_body(in_hbm, masks_hbm, out_hbm, tbl, inbuf, mbuf, red,
             isem0, isem1, msem0, msem1):
    wid = lax.axis_index("s") * 2 + lax.axis_index("c")
    laneoff = lax.iota(jnp.int32, 16) * _NB
    ones = jnp.ones((16,), jnp.float32)
    signbit = jnp.int32(-2147483648)
    isems = (isem0, isem1)
    msems = (msem0, msem1)

    @pl.loop(0, _TPW)
    def _task(t):
        task = wid * _TPW + t
        j = jnp.where(task >= _C, 1, 0)
        c = task - j * _C

        @pl.loop(0, _TBL // 16, unroll=8)
        def _zero(i):
            tbl[pl.ds(i * 16, 16)] = jnp.zeros((16,), jnp.float32)

        def issue(s, b):
            pltpu.async_copy(in_hbm.at[c, pl.ds(s * _CH, _CH)],
                             inbuf.at[b], isems[b])
            pltpu.async_copy(masks_hbm.at[j, pl.ds(s * _CH, _CH)],
                             mbuf.at[b], msems[b])

        def wait(s, b):
            pltpu.make_async_copy(in_hbm.at[c, pl.ds(s * _CH, _CH)],
                                  inbuf.at[b], isems[b]).wait()
            pltpu.make_async_copy(masks_hbm.at[j, pl.ds(s * _CH, _CH)],
                                  mbuf.at[b], msems[b]).wait()

        def compute(b):
            @pl.loop(0, _CH // 16, unroll=4)
            def _vec(i):
                v = inbuf[b, pl.ds(i * 16, 16)]
                m = mbuf[b, pl.ds(i * 16, 16)]
                x = m * v
                bi = lax.bitcast_convert_type(x, jnp.int32)
                sgn = jnp.right_shift(bi, 31)
                key = jnp.bitwise_xor(bi, jnp.bitwise_or(sgn, signbit))
                bkt = lax.shift_right_logical(key, 32 - _KEYBITS)
                idx = laneoff + bkt
                plsc.addupdate_scatter(tbl, [idx], ones)
                plsc.addupdate_scatter(tbl, [idx + _NLANE * _NB], x)
                plsc.addupdate_scatter(tbl, [idx + 2 * _NLANE * _NB], x * x)

        issue(0, 0)

        @pl.loop(0, _NCHUNK // 2)
        def _pair(p):
            s0 = 2 * p
            issue(s0 + 1, 1)
            wait(s0, 0)
            compute(0)

            @pl.when(s0 + 2 < _NCHUNK)
            def _():
                issue(s0 + 2, 0)

            wait(s0 + 1, 1)
            compute(1)

        for a in range(_NARR):
            @pl.loop(0, _NB // 16)
            def _red(i):
                acc = tbl[pl.ds(a * _NLANE * _NB + i * 16, 16)]
                for l in range(1, _NLANE):
                    acc += tbl[pl.ds(a * _NLANE * _NB + l * _NB + i * 16, 16)]
                red[pl.ds(a * _NB + i * 16, 16)] = acc

        pltpu.sync_copy(red, out_hbm.at[task])


@functools.cache
def _sc_hist_kernel():
    return pl.kernel(
        _sc_body,
        out_type=jax.ShapeDtypeStruct((_TASKS, _RED), jnp.float32),
        mesh=plsc.VectorSubcoreMesh(core_axis_name="c", subcore_axis_name="s"),
        scratch_types=[
            pltpu.VMEM((_TBL,), jnp.float32),
            pltpu.VMEM((2, _CH), jnp.float32),
            pltpu.VMEM((2, _CH), jnp.float32),
            pltpu.VMEM((_RED,), jnp.float32),
            pltpu.SemaphoreType.DMA,
            pltpu.SemaphoreType.DMA,
            pltpu.SemaphoreType.DMA,
            pltpu.SemaphoreType.DMA,
        ],
        compiler_params=pltpu.CompilerParams(needs_layout_passes=False),
    )


def _dotg(a, b):
    return lax.dot_general(
        a, b, (((1,), (1,)), ((), ())),
        precision=lax.Precision.HIGHEST,
        preferred_element_type=jnp.float32)


def _fin_body(hist_ref, th_ref, tmin_ref, tmax_ref, u2048_ref, i256_ref,
              u256_ref, out_ref):
    tb = hist_ref[0]
    cnt = tb[0:1, :]
    vsq = tb[2:3, :]
    sumsq = jnp.sum(vsq)

    cc = lax.dot_general(cnt, u2048_ref[...], (((1,), (0,)), ((), ())),
                         precision=lax.Precision.HIGHEST,
                         preferred_element_type=jnp.float32)
    ccp = cc - cnt

    th = th_ref[0]
    cdf = lax.dot_general(th, u256_ref[...], (((1,), (0,)), ((), ())),
                          precision=lax.Precision.HIGHEST,
                          preferred_element_type=jnp.float32)
    total = jnp.maximum(jnp.max(cdf), 1e-12)
    cdfs = cdf / total * jnp.float32(_N)
    i_lane = lax.broadcasted_iota(jnp.int32, (1, _NBINS), 1).astype(jnp.float32)
    r = jnp.floor(cdfs)
    r = jnp.clip(r, 0.0, jnp.float32(_N))
    r = jnp.where(i_lane == jnp.float32(_NBINS - 1), jnp.float32(_N), r)
    rt = _dotg(i256_ref[...], r)

    mlt = (cc < rt).astype(jnp.float32)
    mle = (ccp < rt).astype(jnp.float32)
    below = _dotg(mlt, tb)
    at = _dotg(mle, tb) - below
    below_cnt = below[:, 0:1]
    below_sum = below[:, 1:2]
    at_cnt = at[:, 0:1]
    at_sum = at[:, 1:2]
    at_sq = at[:, 2:3]

    m = rt - below_cnt
    ac = jnp.maximum(at_cnt, 1.0)
    mu = at_sum / ac
    var = jnp.maximum(at_sq / ac - mu * mu, 0.0)
    w = jnp.sqrt(12.0 * var)
    s = below_sum + m * mu - 0.5 * w * m * (1.0 - m / ac)

    tmin = tmin_ref[0, 0, 0]
    tmax = tmax_ref[0, 0, 0]
    i_sub = lax.broadcasted_iota(jnp.int32, (_NBINS, 1), 0).astype(jnp.float32)
    scale = (tmax - tmin) / jnp.float32(_NBINS - 1)
    tv = i_sub * scale + tmin
    last = i_sub == jnp.float32(_NBINS - 1)
    tnext = jnp.where(last, 0.0, (i_sub + 1.0) * scale + tmin)
    tnext2 = jnp.where(last, 0.0, tnext * tnext)
    cross = jnp.sum(s * (tv - tnext))
    sum_t2 = jnp.sum(rt * (tv * tv - tnext2))
    out_ref[0] = (sumsq - 2.0 * cross + sum_t2).reshape(1, 1)


def kernel(input, masks, target_hists, target_mins, target_maxs):
    inp2 = input.reshape(_C, _N)
    m2 = masks.reshape(_J, _N)
    hist = _sc_hist_kernel()(inp2, m2)
    hist3 = hist.reshape(_TASKS, _NARR, _NB)

    th = target_hists.reshape(_TASKS, 1, _NBINS)
    tmin = target_mins.reshape(_TASKS, 1, 1)
    tmax = target_maxs.reshape(_TASKS, 1, 1)

    k2 = lax.broadcasted_iota(jnp.int32, (_NB, _NB), 0)
    b2 = lax.broadcasted_iota(jnp.int32, (_NB, _NB), 1)
    u2048 = (k2 <= b2).astype(jnp.float32)
    k1 = lax.broadcasted_iota(jnp.int32, (_NBINS, _NBINS), 0)
    b1 = lax.broadcasted_iota(jnp.int32, (_NBINS, _NBINS), 1)
    u256 = (k1 <= b1).astype(jnp.float32)
    i256 = (k1 == b1).astype(jnp.float32)

    parts = pl.pallas_call(
        _fin_body,
        grid=(_TASKS,),
        in_specs=[
            pl.BlockSpec((1, _NARR, _NB), lambda i: (i, 0, 0)),
            pl.BlockSpec((1, 1, _NBINS), lambda i: (i, 0, 0)),
            pl.BlockSpec((1, 1, 1), lambda i: (i, 0, 0)),
            pl.BlockSpec((1, 1, 1), lambda i: (i, 0, 0)),
            pl.BlockSpec((_NB, _NB), lambda i: (0, 0)),
            pl.BlockSpec((_NBINS, _NBINS), lambda i: (0, 0)),
            pl.BlockSpec((_NBINS, _NBINS), lambda i: (0, 0)),
        ],
        out_specs=pl.BlockSpec((1, 1, 1), lambda i: (i, 0, 0)),
        out_shape=jax.ShapeDtypeStruct((_TASKS, 1, 1), jnp.float32),
    )(hist3, th, tmin, tmax, u2048, i256, u256)

    return (0.01 / (_C * _N)) * jnp.sum(parts)


# speedup vs baseline: 3017.9087x; 2.1514x over previous
import functools

import jax
import jax.numpy as jnp
from jax import lax
from jax.experimental import pallas as pl
from jax.experimental.pallas import tpu as pltpu
from jax.experimental.pallas import tpu_sc as plsc

_NBINS = 256
_C = 96
_N = 512 * 512
_J = 2
_KEYBITS = 11
_NB = 1 << _KEYBITS
_NLANE = 16
_NARR = 3
_TBL = _NARR * _NLANE * _NB
_TASKS = _J * _C
_NWORK = 32
_TPW = _TASKS // _NWORK
_CH = 4096
_NCHUNK = _N // _CH


def _sqrt(x):
    yi = lax.shift_right_logical(lax.bitcast_convert_type(x, jnp.int32), 1)
    y = lax.bitcast_convert_type(yi + jnp.int32(0x1FBD1DF5), jnp.float32)
    for _ in range(3):
        y = 0.5 * (y + x / y)
    return y


def _sc_body(in_hbm, masks_hbm, th_hbm, tmn_hbm, tmx_hbm, out_hbm,
             tbl, inbuf, mbuf, red, cc, csum, thbuf, tmnbuf, tmxbuf, outbuf,
             isem0, isem1, msem0, msem1):
    wid = lax.axis_index("s") * 2 + lax.axis_index("c")
    laneoff = lax.iota(jnp.int32, 16) * _NB
    lane = lax.iota(jnp.int32, 16)
    ones = jnp.ones((16,), jnp.float32)
    zeros = jnp.zeros((16,), jnp.float32)
    signbit = jnp.int32(-2147483648)
    isems = (isem0, isem1)
    msems = (msem0, msem1)

    @pl.loop(0, _TPW)
    def _task(t):
        task = wid * _TPW + t
        j = jnp.where(task >= _C, 1, 0)
        c = task - j * _C

        pltpu.sync_copy(th_hbm.at[task], thbuf)
        pltpu.sync_copy(tmn_hbm.at[task], tmnbuf)
        pltpu.sync_copy(tmx_hbm.at[task], tmxbuf)

        @pl.loop(0, _TBL // 16, unroll=8)
        def _zero(i):
            tbl[pl.ds(i * 16, 16)] = zeros

        def issue(s, b):
            pltpu.async_copy(in_hbm.at[c, pl.ds(s * _CH, _CH)],
                             inbuf.at[b], isems[b])
            pltpu.async_copy(masks_hbm.at[j, pl.ds(s * _CH, _CH)],
                             mbuf.at[b], msems[b])

        def wait(s, b):
            pltpu.make_async_copy(in_hbm.at[c, pl.ds(s * _CH, _CH)],
                                  inbuf.at[b], isems[b]).wait()
            pltpu.make_async_copy(masks_hbm.at[j, pl.ds(s * _CH, _CH)],
                                  mbuf.at[b], msems[b]).wait()

        def compute(b):
            @pl.loop(0, _CH // 16, unroll=4)
            def _vec(i):
                v = inbuf[b, pl.ds(i * 16, 16)]
                m = mbuf[b, pl.ds(i * 16, 16)]
                x = m * v
                bi = lax.bitcast_convert_type(x, jnp.int32)
                sgn = jnp.right_shift(bi, 31)
                key = jnp.bitwise_xor(bi, jnp.bitwise_or(sgn, signbit))
                bkt = lax.shift_right_logical(key, 32 - _KEYBITS)
                idx = laneoff + bkt
                plsc.addupdate_scatter(tbl, [idx], ones)
                plsc.addupdate_scatter(tbl, [idx + _NLANE * _NB], x)
                plsc.addupdate_scatter(tbl, [idx + 2 * _NLANE * _NB], x * x)

        issue(0, 0)

        @pl.loop(0, _NCHUNK // 2)
        def _pair(p):
            s0 = 2 * p
            issue(s0 + 1, 1)
            wait(s0, 0)
            compute(0)

            @pl.when(s0 + 2 < _NCHUNK)
            def _():
                issue(s0 + 2, 0)

            wait(s0 + 1, 1)
            compute(1)

        for a in range(_NARR):
            @pl.loop(0, _NB // 16)
            def _red(i):
                acc = tbl[pl.ds(a * _NLANE * _NB + i * 16, 16)]
                for l in range(1, _NLANE):
                    acc += tbl[pl.ds(a * _NLANE * _NB + l * _NB + i * 16, 16)]
                red[pl.ds(a * _NB + i * 16, 16)] = acc

        def _cs_body(i, carry):
            cc_c, cs_c, sq_acc = carry
            cv = red[pl.ds(i * 16, 16)]
            sv = red[pl.ds(_NB + i * 16, 16)]
            qv = red[pl.ds(2 * _NB + i * 16, 16)]
            cc[pl.ds(i * 16, 16)] = plsc.cumsum(cv) + cc_c
            csum[pl.ds(i * 16, 16)] = plsc.cumsum(sv) + cs_c
            return (cc_c + jnp.sum(cv), cs_c + jnp.sum(sv), sq_acc + qv)

        _, _, sq_acc = lax.fori_loop(0, _NB // 16, _cs_body,
                                     (zeros, zeros, zeros))

        def _th_body(g, carry):
            hv = thbuf[pl.ds(g * 16, 16)]
            thbuf[pl.ds(g * 16, 16)] = plsc.cumsum(hv) + carry
            return carry + jnp.sum(hv)

        total = lax.fori_loop(0, _NBINS // 16, _th_body, zeros)
        total = jnp.maximum(total, 1e-12)

        tmn = tmnbuf[pl.ds(0, 16)]
        tmx = tmxbuf[pl.ds(0, 16)]
        scale = (tmx - tmn) / jnp.float32(_NBINS - 1)
        nf = jnp.float32(_N)

        cross_acc = zeros
        st2_acc = zeros
        for g in range(_NBINS // 16):
            cdfv = thbuf[pl.ds(g * 16, 16)]
            cdfs = cdfv / total * nf
            r = jnp.clip(cdfs.astype(jnp.int32).astype(jnp.float32), 0.0, nf)
            b_idx = lane + g * 16
            r = jnp.where(b_idx == _NBINS - 1, nf, r)

            base = jnp.zeros((16,), jnp.int32)
            for bit in (1024, 512, 256, 128, 64, 32, 16, 8, 4, 2, 1):
                mid = base + (bit - 1)
                v = plsc.load_gather(cc, [mid])
                base = jnp.where(v < r, base + bit, base)
            k = base
            km1 = jnp.maximum(k - 1, 0)
            zerok = k == 0
            ccm = jnp.where(zerok, 0.0, plsc.load_gather(cc, [km1]))
            csm = jnp.where(zerok, 0.0, plsc.load_gather(csum, [km1]))
            cnt_at = plsc.load_gather(red, [k])
            sum_at = plsc.load_gather(red, [k + _NB])
            sq_at = plsc.load_gather(red, [k + 2 * _NB])

            m = r - ccm
            ac = jnp.maximum(cnt_at, 1.0)
            mu = sum_at / ac
            var = jnp.maximum(sq_at / ac - mu * mu, 0.0)
            w = _sqrt(12.0 * var)
            s = csm + m * mu - 0.5 * w * m * (1.0 - m / ac)

            bf = b_idx.astype(jnp.float32)
            tv = bf * scale + tmn
            lastm = b_idx == _NBINS - 1
            tnext = jnp.where(lastm, 0.0, (bf + 1.0) * scale + tmn)
            tnext2 = jnp.where(lastm, 0.0, tnext * tnext)
            cross_acc = cross_acc + s * (tv - tnext)
            st2_acc = st2_acc + r * (tv * tv - tnext2)

        outbuf[pl.ds(0, 16)] = sq_acc - 2.0 * cross_acc + st2_acc
        pltpu.sync_copy(outbuf, out_hbm.at[task])


@functools.cache
def _sc_kernel():
    return pl.kernel(
        _sc_body,
        out_type=jax.ShapeDtypeStruct((_TASKS, 16), jnp.float32),
        mesh=plsc.VectorSubcoreMesh(core_axis_name="c", subcore_axis_name="s"),
        scratch_types=[
            pltpu.VMEM((_TBL,), jnp.float32),
            pltpu.VMEM((2, _CH), jnp.float32),
            pltpu.VMEM((2, _CH), jnp.float32),
            pltpu.VMEM((_NARR * _NB,), jnp.float32),
            pltpu.VMEM((_NB,), jnp.float32),
            pltpu.VMEM((_NB,), jnp.float32),
            pltpu.VMEM((_NBINS,), jnp.float32),
            pltpu.VMEM((16,), jnp.float32),
            pltpu.VMEM((16,), jnp.float32),
            pltpu.VMEM((16,), jnp.float32),
            pltpu.SemaphoreType.DMA,
            pltpu.SemaphoreType.DMA,
            pltpu.SemaphoreType.DMA,
            pltpu.SemaphoreType.DMA,
        ],
        compiler_params=pltpu.CompilerParams(needs_layout_passes=False),
    )


def kernel(input, masks, target_hists, target_mins, target_maxs):
    inp2 = input.reshape(_C, _N)
    m2 = masks.reshape(_J, _N)
    th2 = target_hists.reshape(_TASKS, _NBINS)
    tmn = jnp.broadcast_to(target_mins.reshape(_TASKS, 1), (_TASKS, 16))
    tmx = jnp.broadcast_to(target_maxs.reshape(_TASKS, 1), (_TASKS, 16))
    parts = _sc_kernel()(inp2, m2, th2, tmn, tmx)
    return (0.01 / (_C * _N)) * jnp.sum(parts)


# speedup vs baseline: 6930.1685x; 2.2963x over previous
import functools

import jax
import jax.numpy as jnp
from jax import lax
from jax.experimental import pallas as pl
from jax.experimental.pallas import tpu as pltpu
from jax.experimental.pallas import tpu_sc as plsc

_NBINS = 256
_C = 96
_N = 512 * 512
_J = 2
_KEYBITS = 11
_NB = 1 << _KEYBITS
_NLANE = 16
_NARR = 3
_TBL = _NARR * _NLANE * _NB
_TASKS = _J * _C
_NWORK = 32
_TPW = _TASKS // _NWORK
_CH = 4096
_NCHUNK = _N // _CH


def _sqrt(x):
    yi = lax.shift_right_logical(lax.bitcast_convert_type(x, jnp.int32), 1)
    y = lax.bitcast_convert_type(yi + jnp.int32(0x1FBD1DF5), jnp.float32)
    for _ in range(3):
        y = 0.5 * (y + x / y)
    return y


def _sc_body(in_hbm, masks_hbm, th_hbm, tmn_hbm, tmx_hbm, out_hbm,
             tbl, inbuf, mbuf, red, cc, csum, thbuf, tmnbuf, tmxbuf, outbuf,
             isem0, isem1, msem0, msem1):
    wid = lax.axis_index("s") * 2 + lax.axis_index("c")
    laneoff = lax.iota(jnp.int32, 16) * _NB
    lane = lax.iota(jnp.int32, 16)
    ones = jnp.ones((16,), jnp.float32)
    zeros = jnp.zeros((16,), jnp.float32)
    signbit = jnp.int32(-2147483648)
    isems = (isem0, isem1)
    msems = (msem0, msem1)

    @pl.loop(0, _TPW)
    def _task(t):
        task = wid * _TPW + t
        j = jnp.where(task >= _C, 1, 0)
        c = task - j * _C

        pltpu.sync_copy(th_hbm.at[task], thbuf)
        pltpu.sync_copy(tmn_hbm.at[task], tmnbuf)
        pltpu.sync_copy(tmx_hbm.at[task], tmxbuf)

        @pl.loop(0, _TBL // 16, unroll=8)
        def _zero(i):
            tbl[pl.ds(i * 16, 16)] = zeros

        def issue(s, b):
            pltpu.async_copy(in_hbm.at[c, pl.ds(s * _CH, _CH)],
                             inbuf.at[b], isems[b])
            pltpu.async_copy(masks_hbm.at[j, pl.ds(s * _CH, _CH)],
                             mbuf.at[b], msems[b])

        def wait(s, b):
            pltpu.make_async_copy(in_hbm.at[c, pl.ds(s * _CH, _CH)],
                                  inbuf.at[b], isems[b]).wait()
            pltpu.make_async_copy(masks_hbm.at[j, pl.ds(s * _CH, _CH)],
                                  mbuf.at[b], msems[b]).wait()

        def compute(b):
            @plsc.parallel_loop(0, _CH // 16, 1, unroll=4)
            def _vec(i):
                v = inbuf[b, pl.ds(i * 16, 16)]
                m = mbuf[b, pl.ds(i * 16, 16)]
                x = m * v
                bi = lax.bitcast_convert_type(x, jnp.int32)
                sgn = jnp.right_shift(bi, 31)
                key = jnp.bitwise_xor(bi, jnp.bitwise_or(sgn, signbit))
                bkt = lax.shift_right_logical(key, 32 - _KEYBITS)
                idx = laneoff + bkt
                plsc.addupdate_scatter(tbl, [idx], ones)
                plsc.addupdate_scatter(tbl, [idx + _NLANE * _NB], x)
                plsc.addupdate_scatter(tbl, [idx + 2 * _NLANE * _NB], x * x)

        issue(0, 0)

        @pl.loop(0, _NCHUNK // 2)
        def _pair(p):
            s0 = 2 * p
            issue(s0 + 1, 1)
            wait(s0, 0)
            compute(0)

            @pl.when(s0 + 2 < _NCHUNK)
            def _():
                issue(s0 + 2, 0)

            wait(s0 + 1, 1)
            compute(1)

        for a in range(_NARR):
            @pl.loop(0, _NB // 16)
            def _red(i):
                acc = tbl[pl.ds(a * _NLANE * _NB + i * 16, 16)]
                for l in range(1, _NLANE):
                    acc += tbl[pl.ds(a * _NLANE * _NB + l * _NB + i * 16, 16)]
                red[pl.ds(a * _NB + i * 16, 16)] = acc

        def _cs_body(i, carry):
            cc_c, cs_c, sq_acc = carry
            cv = red[pl.ds(i * 16, 16)]
            sv = red[pl.ds(_NB + i * 16, 16)]
            qv = red[pl.ds(2 * _NB + i * 16, 16)]
            cc[pl.ds(i * 16, 16)] = plsc.cumsum(cv) + cc_c
            csum[pl.ds(i * 16, 16)] = plsc.cumsum(sv) + cs_c
            return (cc_c + jnp.sum(cv), cs_c + jnp.sum(sv), sq_acc + qv)

        _, _, sq_acc = lax.fori_loop(0, _NB // 16, _cs_body,
                                     (zeros, zeros, zeros))

        def _th_body(g, carry):
            hv = thbuf[pl.ds(g * 16, 16)]
            thbuf[pl.ds(g * 16, 16)] = plsc.cumsum(hv) + carry
            return carry + jnp.sum(hv)

        total = lax.fori_loop(0, _NBINS // 16, _th_body, zeros)
        total = jnp.maximum(total, 1e-12)

        tmn = tmnbuf[pl.ds(0, 16)]
        tmx = tmxbuf[pl.ds(0, 16)]
        scale = (tmx - tmn) / jnp.float32(_NBINS - 1)
        nf = jnp.float32(_N)

        cross_acc = zeros
        st2_acc = zeros
        for g in range(_NBINS // 16):
            cdfv = thbuf[pl.ds(g * 16, 16)]
            cdfs = cdfv / total * nf
            r = jnp.clip(cdfs.astype(jnp.int32).astype(jnp.float32), 0.0, nf)
            b_idx = lane + g * 16
            r = jnp.where(b_idx == _NBINS - 1, nf, r)

            base = jnp.zeros((16,), jnp.int32)
            for bit in (1024, 512, 256, 128, 64, 32, 16, 8, 4, 2, 1):
                mid = base + (bit - 1)
                v = plsc.load_gather(cc, [mid])
                base = jnp.where(v < r, base + bit, base)
            k = base
            km1 = jnp.maximum(k - 1, 0)
            zerok = k == 0
            ccm = jnp.where(zerok, 0.0, plsc.load_gather(cc, [km1]))
            csm = jnp.where(zerok, 0.0, plsc.load_gather(csum, [km1]))
            cnt_at = plsc.load_gather(red, [k])
            sum_at = plsc.load_gather(red, [k + _NB])
            sq_at = plsc.load_gather(red, [k + 2 * _NB])

            m = r - ccm
            ac = jnp.maximum(cnt_at, 1.0)
            mu = sum_at / ac
            var = jnp.maximum(sq_at / ac - mu * mu, 0.0)
            w = _sqrt(12.0 * var)
            s = csm + m * mu - 0.5 * w * m * (1.0 - m / ac)

            bf = b_idx.astype(jnp.float32)
            tv = bf * scale + tmn
            lastm = b_idx == _NBINS - 1
            tnext = jnp.where(lastm, 0.0, (bf + 1.0) * scale + tmn)
            tnext2 = jnp.where(lastm, 0.0, tnext * tnext)
            cross_acc = cross_acc + s * (tv - tnext)
            st2_acc = st2_acc + r * (tv * tv - tnext2)

        outbuf[pl.ds(0, 16)] = sq_acc - 2.0 * cross_acc + st2_acc
        pltpu.sync_copy(outbuf, out_hbm.at[task])


@functools.cache
def _sc_kernel():
    return pl.kernel(
        _sc_body,
        out_type=jax.ShapeDtypeStruct((_TASKS, 16), jnp.float32),
        mesh=plsc.VectorSubcoreMesh(core_axis_name="c", subcore_axis_name="s"),
        scratch_types=[
            pltpu.VMEM((_TBL,), jnp.float32),
            pltpu.VMEM((2, _CH), jnp.float32),
            pltpu.VMEM((2, _CH), jnp.float32),
            pltpu.VMEM((_NARR * _NB,), jnp.float32),
            pltpu.VMEM((_NB,), jnp.float32),
            pltpu.VMEM((_NB,), jnp.float32),
            pltpu.VMEM((_NBINS,), jnp.float32),
            pltpu.VMEM((16,), jnp.float32),
            pltpu.VMEM((16,), jnp.float32),
            pltpu.VMEM((16,), jnp.float32),
            pltpu.SemaphoreType.DMA,
            pltpu.SemaphoreType.DMA,
            pltpu.SemaphoreType.DMA,
            pltpu.SemaphoreType.DMA,
        ],
        compiler_params=pltpu.CompilerParams(needs_layout_passes=False),
    )


def kernel(input, masks, target_hists, target_mins, target_maxs):
    inp2 = input.reshape(_C, _N)
    m2 = masks.reshape(_J, _N)
    th2 = target_hists.reshape(_TASKS, _NBINS)
    tmn = jnp.broadcast_to(target_mins.reshape(_TASKS, 1), (_TASKS, 16))
    tmx = jnp.broadcast_to(target_maxs.reshape(_TASKS, 1), (_TASKS, 16))
    parts = _sc_kernel()(inp2, m2, th2, tmn, tmx)
    return (0.01 / (_C * _N)) * jnp.sum(parts)


# speedup vs baseline: 8480.4028x; 1.2237x over previous
import functools

import numpy as np

import jax
import jax.numpy as jnp
from jax import lax
from jax.experimental import pallas as pl
from jax.experimental.pallas import tpu as pltpu
from jax.experimental.pallas import tpu_sc as plsc

_NBINS = 256
_C = 96
_N = 512 * 512
_J = 2
_KEYBITS = 11
_NB = 1 << _KEYBITS
_NLANE = 16
_NARR = 2
_TBL = _NARR * _NLANE * _NB
_TASKS = _J * _C
_NWORK = 32
_TPW = _TASKS // _NWORK
_CH = 8192
_NCHUNK = _N // _CH


def _width_table():
    keys = np.arange(_NB + 1, dtype=np.uint64)
    b = (keys << (32 - _KEYBITS)).astype(np.uint32)
    neg = (b & np.uint32(0x80000000)) == 0
    mag = np.where(neg, ~b, b & np.uint32(0x7FFFFFFF)).astype(np.uint32)
    expo = (mag >> np.uint32(23)) & np.uint32(0xFF)
    mag = np.where(expo >= 255, np.uint32(0x7F000000), mag).astype(np.uint32)
    v = mag.view(np.float32)
    bnd = np.where(neg, -np.abs(v), np.abs(v))
    return np.abs(bnd[1:] - bnd[:-1]).astype(np.float32)


_WIDTH = _width_table()


def _sc_body(in_hbm, masks_hbm, th_hbm, tmn_hbm, tmx_hbm, wid_hbm, out_hbm,
             tbl, inbuf, mbuf, red, cc, csum, thbuf, tmnbuf, tmxbuf, widbuf,
             outbuf, isem0, isem1, msem0, msem1):
    wid = lax.axis_index("s") * 2 + lax.axis_index("c")
    laneoff = lax.iota(jnp.int32, 16) * _NB
    lane = lax.iota(jnp.int32, 16)
    ones = jnp.ones((16,), jnp.float32)
    zeros = jnp.zeros((16,), jnp.float32)
    signbit = jnp.int32(-2147483648)
    isems = (isem0, isem1)
    msems = (msem0, msem1)

    pltpu.sync_copy(wid_hbm, widbuf)

    @pl.loop(0, _TPW)
    def _task(t):
        task = wid * _TPW + t
        j = jnp.where(task >= _C, 1, 0)
        c = task - j * _C

        pltpu.sync_copy(th_hbm.at[task], thbuf)
        pltpu.sync_copy(tmn_hbm.at[task], tmnbuf)
        pltpu.sync_copy(tmx_hbm.at[task], tmxbuf)

        @pl.loop(0, _TBL // 16, unroll=8)
        def _zero(i):
            tbl[pl.ds(i * 16, 16)] = zeros

        def issue(s, b):
            pltpu.async_copy(in_hbm.at[c, pl.ds(s * _CH, _CH)],
                             inbuf.at[b], isems[b])
            pltpu.async_copy(masks_hbm.at[j, pl.ds(s * _CH, _CH)],
                             mbuf.at[b], msems[b])

        def wait(s, b):
            pltpu.make_async_copy(in_hbm.at[c, pl.ds(s * _CH, _CH)],
                                  inbuf.at[b], isems[b]).wait()
            pltpu.make_async_copy(masks_hbm.at[j, pl.ds(s * _CH, _CH)],
                                  mbuf.at[b], msems[b]).wait()

        def compute(b, carry):
            def one(base, acc):
                v = inbuf[b, pl.ds(base, 16)]
                m = mbuf[b, pl.ds(base, 16)]
                x = m * v
                bi = lax.bitcast_convert_type(x, jnp.int32)
                sgn = jnp.right_shift(bi, 31)
                key = jnp.bitwise_xor(bi, jnp.bitwise_or(sgn, signbit))
                bkt = lax.shift_right_logical(key, 32 - _KEYBITS)
                idx = laneoff + bkt
                plsc.addupdate_scatter(tbl, [idx], ones)
                plsc.addupdate_scatter(tbl, [idx + _NLANE * _NB], x)
                return acc + x * x

            @plsc.parallel_loop(0, _CH // 64, 1, carry=carry)
            def _vec(i, acc):
                a0, a1, a2, a3 = acc
                base = i * 64
                a0 = one(base, a0)
                a1 = one(base + 16, a1)
                a2 = one(base + 32, a2)
                a3 = one(base + 48, a3)
                return (a0, a1, a2, a3)

            return _vec

        acc = (zeros, zeros, zeros, zeros)
        issue(0, 0)

        @pl.loop(0, _NCHUNK // 2, init_carry=acc)
        def _pair(p, acc):
            s0 = 2 * p
            issue(s0 + 1, 1)
            wait(s0, 0)
            acc = compute(0, acc)

            @pl.when(s0 + 2 < _NCHUNK)
            def _():
                issue(s0 + 2, 0)

            wait(s0 + 1, 1)
            return compute(1, acc)

        a0, a1, a2, a3 = _pair
        sq_acc = (a0 + a1) + (a2 + a3)

        for a in range(_NARR):
            @pl.loop(0, _NB // 16)
            def _red(i):
                acc = tbl[pl.ds(a * _NLANE * _NB + i * 16, 16)]
                for l in range(1, _NLANE):
                    acc += tbl[pl.ds(a * _NLANE * _NB + l * _NB + i * 16, 16)]
                red[pl.ds(a * _NB + i * 16, 16)] = acc

        def _cs_body(i, carry):
            cc_c, cs_c = carry
            cv = red[pl.ds(i * 16, 16)]
            sv = red[pl.ds(_NB + i * 16, 16)]
            cc[pl.ds(i * 16, 16)] = plsc.cumsum(cv) + cc_c
            csum[pl.ds(i * 16, 16)] = plsc.cumsum(sv) + cs_c
            return (cc_c + jnp.sum(cv), cs_c + jnp.sum(sv))

        lax.fori_loop(0, _NB // 16, _cs_body, (zeros, zeros))

        def _th_body(g, carry):
            hv = thbuf[pl.ds(g * 16, 16)]
            thbuf[pl.ds(g * 16, 16)] = plsc.cumsum(hv) + carry
            return carry + jnp.sum(hv)

        total = lax.fori_loop(0, _NBINS // 16, _th_body, zeros)
        total = jnp.maximum(total, 1e-12)

        tmn = tmnbuf[pl.ds(0, 16)]
        tmx = tmxbuf[pl.ds(0, 16)]
        scale = (tmx - tmn) / jnp.float32(_NBINS - 1)
        nf = jnp.float32(_N)

        cross_acc = zeros
        st2_acc = zeros
        for g in range(_NBINS // 16):
            cdfv = thbuf[pl.ds(g * 16, 16)]
            cdfs = cdfv / total * nf
            r = jnp.clip(cdfs.astype(jnp.int32).astype(jnp.float32), 0.0, nf)
            b_idx = lane + g * 16
            r = jnp.where(b_idx == _NBINS - 1, nf, r)

            base = jnp.zeros((16,), jnp.int32)
            for bit in (1024, 512, 256, 128, 64, 32, 16, 8, 4, 2, 1):
                mid = base + (bit - 1)
                v = plsc.load_gather(cc, [mid])
                base = jnp.where(v < r, base + bit, base)
            k = base
            km1 = jnp.maximum(k - 1, 0)
            zerok = k == 0
            ccm = jnp.where(zerok, 0.0, plsc.load_gather(cc, [km1]))
            csm = jnp.where(zerok, 0.0, plsc.load_gather(csum, [km1]))
            cnt_at = plsc.load_gather(red, [k])
            sum_at = plsc.load_gather(red, [k + _NB])
            wk = plsc.load_gather(widbuf, [k])

            m = r - ccm
            ac = jnp.maximum(cnt_at, 1.0)
            mu = sum_at / ac
            s = csm + m * mu - 0.5 * wk * m * (1.0 - m / ac)

            bf = b_idx.astype(jnp.float32)
            tv = bf * scale + tmn
            lastm = b_idx == _NBINS - 1
            tnext = jnp.where(lastm, 0.0, (bf + 1.0) * scale + tmn)
            tnext2 = jnp.where(lastm, 0.0, tnext * tnext)
            cross_acc = cross_acc + s * (tv - tnext)
            st2_acc = st2_acc + r * (tv * tv - tnext2)

        outbuf[pl.ds(0, 16)] = sq_acc - 2.0 * cross_acc + st2_acc
        pltpu.sync_copy(outbuf, out_hbm.at[task])


@functools.cache
def _sc_kernel():
    return pl.kernel(
        _sc_body,
        out_type=jax.ShapeDtypeStruct((_TASKS, 16), jnp.float32),
        mesh=plsc.VectorSubcoreMesh(core_axis_name="c", subcore_axis_name="s"),
        scratch_types=[
            pltpu.VMEM((_TBL,), jnp.float32),
            pltpu.VMEM((2, _CH), jnp.float32),
            pltpu.VMEM((2, _CH), jnp.float32),
            pltpu.VMEM((_NARR * _NB,), jnp.float32),
            pltpu.VMEM((_NB,), jnp.float32),
            pltpu.VMEM((_NB,), jnp.float32),
            pltpu.VMEM((_NBINS,), jnp.float32),
            pltpu.VMEM((16,), jnp.float32),
            pltpu.VMEM((16,), jnp.float32),
            pltpu.VMEM((_NB,), jnp.float32),
            pltpu.VMEM((16,), jnp.float32),
            pltpu.SemaphoreType.DMA,
            pltpu.SemaphoreType.DMA,
            pltpu.SemaphoreType.DMA,
            pltpu.SemaphoreType.DMA,
        ],
        compiler_params=pltpu.CompilerParams(needs_layout_passes=False),
    )


def kernel(input, masks, target_hists, target_mins, target_maxs):
    inp2 = input.reshape(_C, _N)
    m2 = masks.reshape(_J, _N)
    th2 = target_hists.reshape(_TASKS, _NBINS)
    tmn = jnp.broadcast_to(target_mins.reshape(_TASKS, 1), (_TASKS, 16))
    tmx = jnp.broadcast_to(target_maxs.reshape(_TASKS, 1), (_TASKS, 16))
    wtab = jnp.asarray(_WIDTH)
    parts = _sc_kernel()(inp2, m2, th2, tmn, tmx, wtab)
    return (0.01 / (_C * _N)) * jnp.sum(parts)


# speedup vs baseline: 8598.9993x; 1.0140x over previous
import functools

import numpy as np

import jax
import jax.numpy as jnp
from jax import lax
from jax.experimental import pallas as pl
from jax.experimental.pallas import tpu as pltpu
from jax.experimental.pallas import tpu_sc as plsc

_NBINS = 256
_C = 96
_N = 512 * 512
_J = 2
_KEYBITS = 11
_NB = 1 << _KEYBITS
_NLANE = 16
_NARR = 2
_TBL = _NARR * _NLANE * _NB
_TASKS = _J * _C
_NWORK = 32
_TPW = _TASKS // _NWORK
_CH = 8192
_NCHUNK = _N // _CH


def _width_table():
    keys = np.arange(_NB + 1, dtype=np.uint64)
    b = (keys << (32 - _KEYBITS)).astype(np.uint32)
    neg = (b & np.uint32(0x80000000)) == 0
    mag = np.where(neg, ~b, b & np.uint32(0x7FFFFFFF)).astype(np.uint32)
    expo = (mag >> np.uint32(23)) & np.uint32(0xFF)
    mag = np.where(expo >= 255, np.uint32(0x7F000000), mag).astype(np.uint32)
    v = mag.view(np.float32)
    bnd = np.where(neg, -np.abs(v), np.abs(v))
    return np.abs(bnd[1:] - bnd[:-1]).astype(np.float32)


_WIDTH = _width_table()


def _sc_body(in_hbm, masks_hbm, th_hbm, tmn_hbm, tmx_hbm, wid_hbm, out_hbm,
             tbl, inbuf, mbuf, red, cc, csum, thbuf, tmnbuf, tmxbuf, widbuf,
             outbuf, isem0, isem1, msem0, msem1):
    wid = lax.axis_index("s") * 2 + lax.axis_index("c")
    laneoff = lax.iota(jnp.int32, 16) * _NB
    lane = lax.iota(jnp.int32, 16)
    ones = jnp.ones((16,), jnp.float32)
    zeros = jnp.zeros((16,), jnp.float32)
    signbit = jnp.int32(-2147483648)
    isems = (isem0, isem1)
    msems = (msem0, msem1)

    pltpu.sync_copy(wid_hbm, widbuf)

    @pl.loop(0, _TPW)
    def _task(t):
        task = wid * _TPW + t
        j = jnp.where(task >= _C, 1, 0)
        c = task - j * _C

        pltpu.sync_copy(th_hbm.at[task], thbuf)
        pltpu.sync_copy(tmn_hbm.at[task], tmnbuf)
        pltpu.sync_copy(tmx_hbm.at[task], tmxbuf)

        @plsc.parallel_loop(0, _TBL // 16, 1, unroll=8)
        def _zero(i):
            tbl[pl.ds(i * 16, 16)] = zeros

        def issue(s, b):
            pltpu.async_copy(in_hbm.at[c, pl.ds(s * _CH, _CH)],
                             inbuf.at[b], isems[b])
            pltpu.async_copy(masks_hbm.at[j, pl.ds(s * _CH, _CH)],
                             mbuf.at[b], msems[b])

        def wait(s, b):
            pltpu.make_async_copy(in_hbm.at[c, pl.ds(s * _CH, _CH)],
                                  inbuf.at[b], isems[b]).wait()
            pltpu.make_async_copy(masks_hbm.at[j, pl.ds(s * _CH, _CH)],
                                  mbuf.at[b], msems[b]).wait()

        def compute(b, carry):
            def one(base, acc):
                v = inbuf[b, pl.ds(base, 16)]
                m = mbuf[b, pl.ds(base, 16)]
                x = m * v
                bi = lax.bitcast_convert_type(x, jnp.int32)
                sgn = jnp.right_shift(bi, 31)
                key = jnp.bitwise_xor(bi, jnp.bitwise_or(sgn, signbit))
                bkt = lax.shift_right_logical(key, 32 - _KEYBITS)
                idx = laneoff + bkt
                plsc.addupdate_scatter(tbl, [idx], ones)
                plsc.addupdate_scatter(tbl, [idx + _NLANE * _NB], x)
                return acc + x * x

            @plsc.parallel_loop(0, _CH // 64, 1, carry=carry)
            def _vec(i, acc):
                a0, a1, a2, a3 = acc
                base = i * 64
                a0 = one(base, a0)
                a1 = one(base + 16, a1)
                a2 = one(base + 32, a2)
                a3 = one(base + 48, a3)
                return (a0, a1, a2, a3)

            return _vec

        acc = (zeros, zeros, zeros, zeros)
        issue(0, 0)

        @pl.loop(0, _NCHUNK // 2, init_carry=acc)
        def _pair(p, acc):
            s0 = 2 * p
            issue(s0 + 1, 1)
            wait(s0, 0)
            acc = compute(0, acc)

            @pl.when(s0 + 2 < _NCHUNK)
            def _():
                issue(s0 + 2, 0)

            wait(s0 + 1, 1)
            return compute(1, acc)

        a0, a1, a2, a3 = _pair
        sq_acc = (a0 + a1) + (a2 + a3)

        for a in range(_NARR):
            @plsc.parallel_loop(0, _NB // 16, 1)
            def _red(i):
                acc = tbl[pl.ds(a * _NLANE * _NB + i * 16, 16)]
                for l in range(1, _NLANE):
                    acc += tbl[pl.ds(a * _NLANE * _NB + l * _NB + i * 16, 16)]
                red[pl.ds(a * _NB + i * 16, 16)] = acc

        def _cs_body(i, carry):
            cc_c, cs_c = carry
            cv = red[pl.ds(i * 16, 16)]
            sv = red[pl.ds(_NB + i * 16, 16)]
            cc[pl.ds(i * 16, 16)] = plsc.cumsum(cv) + cc_c
            csum[pl.ds(i * 16, 16)] = plsc.cumsum(sv) + cs_c
            return (cc_c + jnp.sum(cv), cs_c + jnp.sum(sv))

        lax.fori_loop(0, _NB // 16, _cs_body, (zeros, zeros))

        def _th_body(g, carry):
            hv = thbuf[pl.ds(g * 16, 16)]
            thbuf[pl.ds(g * 16, 16)] = plsc.cumsum(hv) + carry
            return carry + jnp.sum(hv)

        total = lax.fori_loop(0, _NBINS // 16, _th_body, zeros)
        total = jnp.maximum(total, 1e-12)

        tmn = tmnbuf[pl.ds(0, 16)]
        tmx = tmxbuf[pl.ds(0, 16)]
        scale = (tmx - tmn) / jnp.float32(_NBINS - 1)
        nf = jnp.float32(_N)

        cross_acc = zeros
        st2_acc = zeros
        for g in range(_NBINS // 16):
            cdfv = thbuf[pl.ds(g * 16, 16)]
            cdfs = cdfv / total * nf
            r = jnp.clip(cdfs.astype(jnp.int32).astype(jnp.float32), 0.0, nf)
            b_idx = lane + g * 16
            r = jnp.where(b_idx == _NBINS - 1, nf, r)

            base = jnp.zeros((16,), jnp.int32)
            for bit in (1024, 512, 256, 128, 64, 32, 16, 8, 4, 2, 1):
                mid = base + (bit - 1)
                v = plsc.load_gather(cc, [mid])
                base = jnp.where(v < r, base + bit, base)
            k = base
            km1 = jnp.maximum(k - 1, 0)
            zerok = k == 0
            ccm = jnp.where(zerok, 0.0, plsc.load_gather(cc, [km1]))
            csm = jnp.where(zerok, 0.0, plsc.load_gather(csum, [km1]))
            cnt_at = plsc.load_gather(red, [k])
            sum_at = plsc.load_gather(red, [k + _NB])
            wk = plsc.load_gather(widbuf, [k])

            m = r - ccm
            ac = jnp.maximum(cnt_at, 1.0)
            mu = sum_at / ac
            s = csm + m * mu - 0.5 * wk * m * (1.0 - m / ac)

            bf = b_idx.astype(jnp.float32)
            tv = bf * scale + tmn
            lastm = b_idx == _NBINS - 1
            tnext = jnp.where(lastm, 0.0, (bf + 1.0) * scale + tmn)
            tnext2 = jnp.where(lastm, 0.0, tnext * tnext)
            cross_acc = cross_acc + s * (tv - tnext)
            st2_acc = st2_acc + r * (tv * tv - tnext2)

        outbuf[pl.ds(0, 16)] = sq_acc - 2.0 * cross_acc + st2_acc
        pltpu.sync_copy(outbuf, out_hbm.at[task])


@functools.cache
def _sc_kernel():
    return pl.kernel(
        _sc_body,
        out_type=jax.ShapeDtypeStruct((_TASKS, 16), jnp.float32),
        mesh=plsc.VectorSubcoreMesh(core_axis_name="c", subcore_axis_name="s"),
        scratch_types=[
            pltpu.VMEM((_TBL,), jnp.float32),
            pltpu.VMEM((2, _CH), jnp.float32),
            pltpu.VMEM((2, _CH), jnp.float32),
            pltpu.VMEM((_NARR * _NB,), jnp.float32),
            pltpu.VMEM((_NB,), jnp.float32),
            pltpu.VMEM((_NB,), jnp.float32),
            pltpu.VMEM((_NBINS,), jnp.float32),
            pltpu.VMEM((16,), jnp.float32),
            pltpu.VMEM((16,), jnp.float32),
            pltpu.VMEM((_NB,), jnp.float32),
            pltpu.VMEM((16,), jnp.float32),
            pltpu.SemaphoreType.DMA,
            pltpu.SemaphoreType.DMA,
            pltpu.SemaphoreType.DMA,
            pltpu.SemaphoreType.DMA,
        ],
        compiler_params=pltpu.CompilerParams(needs_layout_passes=False),
    )


def kernel(input, masks, target_hists, target_mins, target_maxs):
    inp2 = input.reshape(_C, _N)
    m2 = masks.reshape(_J, _N)
    th2 = target_hists.reshape(_TASKS, _NBINS)
    tmn = jnp.broadcast_to(target_mins.reshape(_TASKS, 1), (_TASKS, 16))
    tmx = jnp.broadcast_to(target_maxs.reshape(_TASKS, 1), (_TASKS, 16))
    wtab = jnp.asarray(_WIDTH)
    parts = _sc_kernel()(inp2, m2, th2, tmn, tmx, wtab)
    return (0.01 / (_C * _N)) * jnp.sum(parts)
